# baseline (speedup 1.0000x reference)
import jax
import jax.numpy as jnp
from jax import lax
from jax.experimental import pallas as pl
from jax.experimental.pallas import tpu as pltpu
from jax.experimental.pallas import tpu_sc as plsc

B, S, D = 4, 2048, 1024
N = B * S
L = 16
NC, NS = 2, 16
NW = NC * NS
TPW = N // NW
G = 16
NG = TPW // G
DV = D // L
GB = 32
NB = TPW // GB
U = 8
PAD = 17

_EPS = 1e-8
_TINY = 1e-30


def _rsqrt_newton(x):
    i = lax.bitcast_convert_type(x, jnp.int32)
    i = jnp.int32(0x5F3759DF) - lax.shift_right_arithmetic(i, 1)
    y = lax.bitcast_convert_type(i, jnp.float32)
    for _ in range(3):
        y = y * (1.5 - 0.5 * x * y * y)
    return y


_DNUMS = lax.GatherDimensionNumbers(
    offset_dims=(), collapsed_slice_dims=(0,), start_index_map=(0,))


def _perm(x, idx):
    return lax.gather(x, idx[:, None], _DNUMS, slice_sizes=(1,),
                      mode=lax.GatherScatterMode.PROMISE_IN_BOUNDS)


def _combine(x, y, k, lane):
    m = (lane & k) == 0
    kv = jnp.full((L,), k, jnp.int32)
    return jnp.where(m, x + _perm(x, lane ^ kv), y + _perm(y, lane ^ kv))


def _body(emitted_hbm, ids_hbm, vocab_hbm, out_hbm,
          ids_v, g_buf0, g_buf1, e_buf0, e_buf1, res_v,
          gsem0, gsem1, esem0, esem1):
    wid = lax.axis_index("s") * NC + lax.axis_index("c")
    base = wid * TPW
    pltpu.sync_copy(ids_hbm.at[pl.ds(base, TPW)], ids_v)

    lane = lax.iota(jnp.int32, L)
    gbufs = ((g_buf0, gsem0), (g_buf1, gsem1))
    ebufs = ((e_buf0, esem0), (e_buf1, esem1))

    def start_g(b, slot):
        gb, gs = gbufs[slot]
        goff = pl.multiple_of(b * GB, GB)
        pltpu.async_copy(vocab_hbm.at[ids_v.at[pl.ds(goff, GB)]], gb, gs)

    def wait_g(slot):
        gb, gs = gbufs[slot]
        pltpu.make_async_copy(
            vocab_hbm.at[ids_v.at[pl.ds(0, GB)]], gb, gs).wait()

    def start_e(h, slot):
        ebf, es = ebufs[slot]
        tok = pl.multiple_of(base + h * G, G)
        pltpu.async_copy(emitted_hbm.at[pl.ds(tok, G)], ebf, es)

    def wait_e(slot):
        ebf, es = ebufs[slot]
        pltpu.make_async_copy(emitted_hbm.at[pl.ds(0, G)], ebf, es).wait()

    def compute(grp, eb, gb, grow, acc_d, acc_m):
        ks = (8, 4, 2, 1)
        pend = {}
        final = None
        for tp in range(0, G, 4):
            def dim_step(jj, accs):
                accs = list(accs)
                for ti in range(4):
                    a_d, a_e, a_g = accs[3 * ti:3 * ti + 3]
                    for k in range(U // 4):
                        off = pl.multiple_of(jj * (L * U // 4) + k * L, L)
                        e = eb[tp + ti, pl.ds(off, L)]
                        g = gb[grow + tp + ti, pl.ds(off, L)]
                        a_d = a_d + e * g
                        a_e = a_e + e * e
                        a_g = a_g + g * g
                    accs[3 * ti:3 * ti + 3] = [a_d, a_e, a_g]
                return tuple(accs)
            zero = jnp.zeros((L,), jnp.float32)
            accs = lax.fori_loop(0, DV // (U // 4), dim_step, (zero,) * 12)
            p0 = tuple(_combine(px, vx, ks[0], lane)
                       for px, vx in zip(accs[:3], accs[3:6]))
            p1 = tuple(_combine(px, vx, ks[0], lane)
                       for px, vx in zip(accs[6:9], accs[9:12]))
            v = tuple(_combine(px, vx, ks[1], lane)
                      for px, vx in zip(p0, p1))
            lvl = 2
            while lvl in pend:
                prev = pend.pop(lvl)
                v = tuple(_combine(px, vx, ks[lvl], lane)
                          for px, vx in zip(prev, v))
                lvl += 1
            if lvl == 4:
                final = v
            else:
                pend[lvl] = v
        dvec, evec, gvec = final

        ids_vec = ids_v[pl.ds(pl.multiple_of(grp * G, G), G)]
        sigma = (lax.shift_left(lane & 1, 3) | lax.shift_left(lane & 2, 1)
                 | lax.shift_right_logical(lane & 4, 1)
                 | lax.shift_right_logical(lane & 8, 3))
        ids_vec = _perm(ids_vec, sigma)
        m = jnp.where(ids_vec != 0, 1.0, 0.0).astype(jnp.float32)
        n1 = jnp.maximum(evec * _rsqrt_newton(jnp.maximum(evec, _TINY)), _EPS)
        n2 = jnp.maximum(gvec * _rsqrt_newton(jnp.maximum(gvec, _TINY)), _EPS)
        dist = 1.0 - dvec / (n1 * n2)
        return acc_d + dist * m, acc_m + m

    def batch(b, gslot, acc_d, acc_m):
        start_e(2 * b + 1, 1)
        wait_g(gslot)
        wait_e(0)
        gb = gbufs[gslot][0]
        eb0 = ebufs[0][0]
        acc_d, acc_m = compute(2 * b, eb0, gb, 0, acc_d, acc_m)

        @pl.when(b < NB - 1)
        def _pref_e0():
            start_e(2 * b + 2, 0)

        wait_e(1)
        eb1 = ebufs[1][0]
        acc_d, acc_m = compute(2 * b + 1, eb1, gb, G, acc_d, acc_m)
        return acc_d, acc_m

    start_g(0, 0)
    start_e(0, 0)

    def pair_step(i, carry):
        acc_d, acc_m = carry

        @pl.when(2 * i + 1 < NB)
        def _pref_g1():
            start_g(2 * i + 1, 1)

        acc_d, acc_m = batch(2 * i, 0, acc_d, acc_m)

        @pl.when(2 * i + 2 < NB)
        def _pref_g0():
            start_g(2 * i + 2, 0)

        acc_d, acc_m = batch(2 * i + 1, 1, acc_d, acc_m)
        return acc_d, acc_m

    zero = jnp.zeros((L,), jnp.float32)
    acc_d, acc_m = lax.fori_loop(0, NB // 2, pair_step, (zero, zero))
    res_v[0, :] = acc_d
    res_v[1, :] = acc_m
    pltpu.sync_copy(res_v, out_hbm.at[wid])


@jax.jit
def _sc_loss(emitted, ids, vocab):
    mesh = plsc.VectorSubcoreMesh(core_axis_name="c", subcore_axis_name="s")
    run = pl.kernel(
        _body,
        out_type=jax.ShapeDtypeStruct((NW, 2, L), jnp.float32),
        mesh=mesh,
        scratch_types=[
            pltpu.VMEM((TPW,), jnp.int32),
            pltpu.VMEM((GB, D), jnp.float32),
            pltpu.VMEM((GB, D), jnp.float32),
            pltpu.VMEM((G, D), jnp.float32),
            pltpu.VMEM((G, D), jnp.float32),
            pltpu.VMEM((2, L), jnp.float32),
            pltpu.SemaphoreType.DMA,
            pltpu.SemaphoreType.DMA,
            pltpu.SemaphoreType.DMA,
            pltpu.SemaphoreType.DMA,
        ],
    )
    partials = run(emitted, ids, vocab)
    return partials[:, 0].sum() / partials[:, 1].sum()


def kernel(emitted_embeddings, target_ids, vocab_basis):
    emitted = emitted_embeddings.reshape(N, D)
    ids = target_ids.reshape(N).astype(jnp.int32)
    return _sc_loss(emitted, ids, vocab_basis)


# speedup vs baseline: 1.1011x; 1.1011x over previous
import jax
import jax.numpy as jnp
from jax import lax
from jax.experimental import pallas as pl
from jax.experimental.pallas import tpu as pltpu
from jax.experimental.pallas import tpu_sc as plsc

B, S, D = 4, 2048, 1024
N = B * S
L = 16
NC, NS = 2, 16
NW = NC * NS
TPW = N // NW
G = 16
NG = TPW // G
DV = D // L
U = 8
PAD = 17

_EPS = 1e-8
_TINY = 1e-30


def _rsqrt_newton(x):
    i = lax.bitcast_convert_type(x, jnp.int32)
    i = jnp.int32(0x5F3759DF) - lax.shift_right_arithmetic(i, 1)
    y = lax.bitcast_convert_type(i, jnp.float32)
    for _ in range(3):
        y = y * (1.5 - 0.5 * x * y * y)
    return y


_DNUMS = lax.GatherDimensionNumbers(
    offset_dims=(), collapsed_slice_dims=(0,), start_index_map=(0,))


def _perm(x, idx):
    return lax.gather(x, idx[:, None], _DNUMS, slice_sizes=(1,),
                      mode=lax.GatherScatterMode.PROMISE_IN_BOUNDS)


def _combine(x, y, k, lane):
    m = (lane & k) == 0
    kv = jnp.full((L,), k, jnp.int32)
    return jnp.where(m, x + _perm(x, lane ^ kv), y + _perm(y, lane ^ kv))


def _body(emitted_hbm, ids_hbm, vocab_hbm, out_hbm,
          ids_v, e_buf0, g_buf0, e_buf1, g_buf1, res_v,
          esem0, gsem0, esem1, gsem1):
    wid = lax.axis_index("s") * NC + lax.axis_index("c")
    base = wid * TPW
    pltpu.sync_copy(ids_hbm.at[pl.ds(base, TPW)], ids_v)

    lane = lax.iota(jnp.int32, L)
    bufs = ((e_buf0, g_buf0, esem0, gsem0), (e_buf1, g_buf1, esem1, gsem1))

    def start(grp, slot):
        eb, gb, es, gs = bufs[slot]
        goff = pl.multiple_of(grp * G, G)
        tok = pl.multiple_of(base + grp * G, G)
        pltpu.async_copy(vocab_hbm.at[ids_v.at[pl.ds(goff, G)]], gb, gs)
        pltpu.async_copy(emitted_hbm.at[pl.ds(tok, G)], eb, es)

    def wait(slot):
        eb, gb, es, gs = bufs[slot]
        pltpu.make_async_copy(emitted_hbm.at[pl.ds(0, G)], eb, es).wait()
        pltpu.make_async_copy(
            vocab_hbm.at[ids_v.at[pl.ds(0, G)]], gb, gs).wait()

    def compute(grp, slot, acc_d, acc_m):
        eb, gb = bufs[slot][0], bufs[slot][1]
        ks = (8, 4, 2, 1)
        pend = {}
        final = None
        for tp in range(0, G, 4):
            def dim_step(jj, accs):
                accs = list(accs)
                for ti in range(4):
                    a_d, a_e, a_g = accs[3 * ti:3 * ti + 3]
                    for k in range(U // 4):
                        off = pl.multiple_of(jj * (L * U // 4) + k * L, L)
                        e = eb[tp + ti, pl.ds(off, L)]
                        g = gb[tp + ti, pl.ds(off, L)]
                        a_d = a_d + e * g
                        a_e = a_e + e * e
                        a_g = a_g + g * g
                    accs[3 * ti:3 * ti + 3] = [a_d, a_e, a_g]
                return tuple(accs)
            zero = jnp.zeros((L,), jnp.float32)
            accs = lax.fori_loop(0, DV // (U // 4), dim_step, (zero,) * 12)
            p0 = tuple(_combine(px, vx, ks[0], lane)
                       for px, vx in zip(accs[:3], accs[3:6]))
            p1 = tuple(_combine(px, vx, ks[0], lane)
                       for px, vx in zip(accs[6:9], accs[9:12]))
            v = tuple(_combine(px, vx, ks[1], lane)
                      for px, vx in zip(p0, p1))
            lvl = 2
            while lvl in pend:
                prev = pend.pop(lvl)
                v = tuple(_combine(px, vx, ks[lvl], lane)
                          for px, vx in zip(prev, v))
                lvl += 1
            if lvl == 4:
                final = v
            else:
                pend[lvl] = v
        dvec, evec, gvec = final

        ids_vec = ids_v[pl.ds(pl.multiple_of(grp * G, G), G)]
        sigma = (lax.shift_left(lane & 1, 3) | lax.shift_left(lane & 2, 1)
                 | lax.shift_right_logical(lane & 4, 1)
                 | lax.shift_right_logical(lane & 8, 3))
        ids_vec = _perm(ids_vec, sigma)
        m = jnp.where(ids_vec != 0, 1.0, 0.0).astype(jnp.float32)
        n1 = jnp.maximum(evec * _rsqrt_newton(jnp.maximum(evec, _TINY)), _EPS)
        n2 = jnp.maximum(gvec * _rsqrt_newton(jnp.maximum(gvec, _TINY)), _EPS)
        dist = 1.0 - dvec / (n1 * n2)
        return acc_d + dist * m, acc_m + m

    start(0, 0)

    def pair_step(i, carry):
        acc_d, acc_m = carry
        start(2 * i + 1, 1)
        wait(0)
        acc_d, acc_m = compute(2 * i, 0, acc_d, acc_m)

        @pl.when(i < NG // 2 - 1)
        def _prefetch():
            start(2 * i + 2, 0)

        wait(1)
        acc_d, acc_m = compute(2 * i + 1, 1, acc_d, acc_m)
        return acc_d, acc_m

    zero = jnp.zeros((L,), jnp.float32)
    acc_d, acc_m = lax.fori_loop(0, NG // 2, pair_step, (zero, zero))
    res_v[0, :] = acc_d
    res_v[1, :] = acc_m
    pltpu.sync_copy(res_v, out_hbm.at[wid])


@jax.jit
def _sc_loss(emitted, ids, vocab):
    mesh = plsc.VectorSubcoreMesh(core_axis_name="c", subcore_axis_name="s")
    run = pl.kernel(
        _body,
        out_type=jax.ShapeDtypeStruct((NW, 2, L), jnp.float32),
        mesh=mesh,
        scratch_types=[
            pltpu.VMEM((TPW,), jnp.int32),
            pltpu.VMEM((G, D), jnp.float32),
            pltpu.VMEM((G, D), jnp.float32),
            pltpu.VMEM((G, D), jnp.float32),
            pltpu.VMEM((G, D), jnp.float32),
            pltpu.VMEM((2, L), jnp.float32),
            pltpu.SemaphoreType.DMA,
            pltpu.SemaphoreType.DMA,
            pltpu.SemaphoreType.DMA,
            pltpu.SemaphoreType.DMA,
        ],
    )
    partials = run(emitted, ids, vocab)
    return partials[:, 0].sum() / partials[:, 1].sum()


def kernel(emitted_embeddings, target_ids, vocab_basis):
    emitted = emitted_embeddings.reshape(N, D)
    ids = target_ids.reshape(N).astype(jnp.int32)
    return _sc_loss(emitted, ids, vocab_basis)


# speedup vs baseline: 1.1218x; 1.0188x over previous
import jax
import jax.numpy as jnp
from jax import lax
from jax.experimental import pallas as pl
from jax.experimental.pallas import tpu as pltpu
from jax.experimental.pallas import tpu_sc as plsc

B, S, D = 4, 2048, 1024
N = B * S
L = 16
NC, NS = 2, 16
NW = NC * NS
TPW = N // NW
G = 16
NG = TPW // G
DV = D // L
U = 8
PAD = 17

_EPS = 1e-8
_TINY = 1e-30


def _rsqrt_newton(x):
    i = lax.bitcast_convert_type(x, jnp.int32)
    i = jnp.int32(0x5F3759DF) - lax.shift_right_arithmetic(i, 1)
    y = lax.bitcast_convert_type(i, jnp.float32)
    for _ in range(3):
        y = y * (1.5 - 0.5 * x * y * y)
    return y


_DNUMS = lax.GatherDimensionNumbers(
    offset_dims=(), collapsed_slice_dims=(0,), start_index_map=(0,))


def _perm(x, idx):
    return lax.gather(x, idx[:, None], _DNUMS, slice_sizes=(1,),
                      mode=lax.GatherScatterMode.PROMISE_IN_BOUNDS)


def _combine(x, y, k, lane):
    m = (lane & k) == 0
    kv = jnp.full((L,), k, jnp.int32)
    return jnp.where(m, x + _perm(x, lane ^ kv), y + _perm(y, lane ^ kv))


def _body(emitted_hbm, ids_hbm, vocab_hbm, out_hbm,
          ids_v, e_buf0, g_buf0, e_buf1, g_buf1, res_v,
          esem0, gsem0, esem1, gsem1):
    wid = lax.axis_index("s") * NC + lax.axis_index("c")
    base = wid * TPW
    pltpu.sync_copy(ids_hbm.at[pl.ds(base, TPW)], ids_v)

    lane = lax.iota(jnp.int32, L)
    bufs = ((e_buf0, g_buf0, esem0, gsem0), (e_buf1, g_buf1, esem1, gsem1))

    def start(grp, slot):
        eb, gb, es, gs = bufs[slot]
        goff = pl.multiple_of(grp * G, G)
        tok = pl.multiple_of(base + grp * G, G)
        pltpu.async_copy(vocab_hbm.at[ids_v.at[pl.ds(goff, G)]], gb, gs)
        pltpu.async_copy(emitted_hbm.at[pl.ds(tok, G)], eb, es)

    def wait(slot):
        eb, gb, es, gs = bufs[slot]
        pltpu.make_async_copy(emitted_hbm.at[pl.ds(0, G)], eb, es).wait()
        pltpu.make_async_copy(
            vocab_hbm.at[ids_v.at[pl.ds(0, G)]], gb, gs).wait()

    def compute(grp, slot, acc_d, acc_m):
        eb, gb = bufs[slot][0], bufs[slot][1]
        ks = (8, 4, 2, 1)
        pend = {}
        final = None
        for tp in range(0, G, 8):
            def dim_step(jj, accs):
                accs = list(accs)
                for ti in range(8):
                    a_d, a_e, a_g = accs[3 * ti:3 * ti + 3]
                    for k in range(U // 8):
                        off = pl.multiple_of(jj * (L * U // 8) + k * L, L)
                        e = eb[tp + ti, pl.ds(off, L)]
                        g = gb[tp + ti, pl.ds(off, L)]
                        a_d = a_d + e * g
                        a_e = a_e + e * e
                        a_g = a_g + g * g
                    accs[3 * ti:3 * ti + 3] = [a_d, a_e, a_g]
                return tuple(accs)
            zero = jnp.zeros((L,), jnp.float32)
            accs = lax.fori_loop(0, DV // (U // 8), dim_step, (zero,) * 24)
            quads = []
            for q in range(4):
                quads.append(tuple(
                    _combine(px, vx, ks[0], lane)
                    for px, vx in zip(accs[6 * q:6 * q + 3],
                                      accs[6 * q + 3:6 * q + 6])))
            h0 = tuple(_combine(px, vx, ks[1], lane)
                       for px, vx in zip(quads[0], quads[1]))
            h1 = tuple(_combine(px, vx, ks[1], lane)
                       for px, vx in zip(quads[2], quads[3]))
            v = tuple(_combine(px, vx, ks[2], lane)
                      for px, vx in zip(h0, h1))
            lvl = 3
            while lvl in pend:
                prev = pend.pop(lvl)
                v = tuple(_combine(px, vx, ks[lvl], lane)
                          for px, vx in zip(prev, v))
                lvl += 1
            if lvl == 4:
                final = v
            else:
                pend[lvl] = v
        dvec, evec, gvec = final

        ids_vec = ids_v[pl.ds(pl.multiple_of(grp * G, G), G)]
        sigma = (lax.shift_left(lane & 1, 3) | lax.shift_left(lane & 2, 1)
                 | lax.shift_right_logical(lane & 4, 1)
                 | lax.shift_right_logical(lane & 8, 3))
        ids_vec = _perm(ids_vec, sigma)
        m = jnp.where(ids_vec != 0, 1.0, 0.0).astype(jnp.float32)
        n1 = jnp.maximum(evec * _rsqrt_newton(jnp.maximum(evec, _TINY)), _EPS)
        n2 = jnp.maximum(gvec * _rsqrt_newton(jnp.maximum(gvec, _TINY)), _EPS)
        dist = 1.0 - dvec / (n1 * n2)
        return acc_d + dist * m, acc_m + m

    start(0, 0)

    def pair_step(i, carry):
        acc_d, acc_m = carry
        start(2 * i + 1, 1)
        wait(0)
        acc_d, acc_m = compute(2 * i, 0, acc_d, acc_m)

        @pl.when(i < NG // 2 - 1)
        def _prefetch():
            start(2 * i + 2, 0)

        wait(1)
        acc_d, acc_m = compute(2 * i + 1, 1, acc_d, acc_m)
        return acc_d, acc_m

    zero = jnp.zeros((L,), jnp.float32)
    acc_d, acc_m = lax.fori_loop(0, NG // 2, pair_step, (zero, zero))
    res_v[0, :] = acc_d
    res_v[1, :] = acc_m
    pltpu.sync_copy(res_v, out_hbm.at[wid])


@jax.jit
def _sc_loss(emitted, ids, vocab):
    mesh = plsc.VectorSubcoreMesh(core_axis_name="c", subcore_axis_name="s")
    run = pl.kernel(
        _body,
        out_type=jax.ShapeDtypeStruct((NW, 2, L), jnp.float32),
        mesh=mesh,
        scratch_types=[
            pltpu.VMEM((TPW,), jnp.int32),
            pltpu.VMEM((G, D), jnp.float32),
            pltpu.VMEM((G, D), jnp.float32),
            pltpu.VMEM((G, D), jnp.float32),
            pltpu.VMEM((G, D), jnp.float32),
            pltpu.VMEM((2, L), jnp.float32),
            pltpu.SemaphoreType.DMA,
            pltpu.SemaphoreType.DMA,
            pltpu.SemaphoreType.DMA,
            pltpu.SemaphoreType.DMA,
        ],
    )
    partials = run(emitted, ids, vocab)
    return partials[:, 0].sum() / partials[:, 1].sum()


def kernel(emitted_embeddings, target_ids, vocab_basis):
    emitted = emitted_embeddings.reshape(N, D)
    ids = target_ids.reshape(N).astype(jnp.int32)
    return _sc_loss(emitted, ids, vocab_basis)
